# baseline (speedup 1.0000x reference)
import functools

import jax
import jax.numpy as jnp
from jax import lax
from jax.experimental import pallas as pl
from jax.experimental.pallas import tpu as pltpu
from jax.experimental.pallas import tpu_sc as plsc

H = 1024
F = 8192
NT = 2
KTOP = 983
N_TOK = 2048
BM_ENC = 256
BM_DEC = 256
BN_DEC = 1024
NCHUNK = 2
RCHUNK = N_TOK // NCHUNK
INT_MIN32 = -2147483648
L = 16
NV = F // L
NW = 32
NBKT = 256


def _enc_body(x_ref, g_ref, bt_ref, w_ref, be_ref, zp_ref):
    x = x_ref[...]
    mean = jnp.mean(x, axis=1, keepdims=True)
    xc = x - mean
    var = jnp.mean(xc * xc, axis=1, keepdims=True)
    xn = xc * jax.lax.rsqrt(var + 1e-5)
    xn = xn * g_ref[...] + bt_ref[...]
    xh = xn.astype(jnp.bfloat16)
    zp = jnp.dot(xh, w_ref[...], preferred_element_type=jnp.float32)
    zp_ref[...] = zp + be_ref[...]


def _dec_body(z_ref, w_ref, bd_ref, y_ref):
    zb16 = z_ref[...].astype(jnp.bfloat16)
    y = jnp.dot(zb16, w_ref[...], preferred_element_type=jnp.float32)
    y_ref[...] = y + bd_ref[...]


_GDN = lax.GatherDimensionNumbers(offset_dims=(), collapsed_slice_dims=(0,),
                                 start_index_map=(0,))


def _splat(x, idx):
    return lax.gather(x, idx[:, None], _GDN, slice_sizes=(1,),
                      mode=lax.GatherScatterMode.PROMISE_IN_BOUNDS)


def _key_of(zb):
    return zb ^ (lax.shift_right_arithmetic(zb, 31) & jnp.int32(0x7FFFFFFF))


def _make_sc_select(chunk_base):
    def body(zp_hbm, z_hbm, row_v, out_v, key_v, hist_v, tot_v, comp_v,
             comp2_v):
        wid = lax.axis_index("s") * 2 + lax.axis_index("c")
        rows = RCHUNK // NW
        base = chunk_base + wid * rows
        iota = lax.iota(jnp.int32, L)
        ones_i = jnp.ones((L,), jnp.int32)
        zeros_i = jnp.zeros((L,), jnp.int32)
        izero = jnp.zeros((L,), jnp.int32)
        ilast = jnp.full((L,), jnp.int32(L - 1))

        def zero_hist(j, c):
            hist_v[pl.ds(j * L, L)] = zeros_i
            return c

        lax.fori_loop(0, (NBKT * L) // L, zero_hist, 0)

        RB = 4

        def do_batch(g, carry0):
            pltpu.sync_copy(zp_hbm.at[pl.ds((base + g * RB) * F, RB * F)],
                            row_v)
            for rb in range(RB):
                _one_row(g, rb)
            pltpu.sync_copy(
                out_v, z_hbm.at[pl.ds((wid * rows + g * RB) * F, RB * F)])
            return carry0

        def _one_row(g, rb):
            rowoff = rb * F

            def p1(j, c):
                for u in range(8):
                    i = j * 8 + u
                    v = row_v[pl.ds(rowoff + i * L, L)]
                    key = _key_of(lax.bitcast_convert_type(v, jnp.int32))
                    key_v[pl.ds(i * L, L)] = key
                    bkt = lax.shift_right_logical(
                        key ^ jnp.int32(INT_MIN32), 24)
                    plsc.addupdate_scatter(hist_v, [bkt + iota * NBKT],
                                           ones_i)
                return c

            lax.fori_loop(0, NV // 8, p1, 0)

            def scan(rank):
                carry = jnp.zeros((L,), jnp.int32)
                bsel = jnp.full((L,), jnp.int32(-1))
                for j in range(15, -1, -1):
                    tot = hist_v[pl.ds(j * L, L)]
                    hist_v[pl.ds(j * L, L)] = zeros_i
                    for s in range(1, L):
                        off = s * NBKT + j * L
                        tot = tot + hist_v[pl.ds(off, L)]
                        hist_v[pl.ds(off, L)] = zeros_i
                    tot_v[pl.ds(j * L, L)] = tot
                    sfx = lax.rev(jnp.cumsum(lax.rev(tot, (0,))), (0,))
                    csum = _splat(sfx, izero)
                    cond = (sfx + carry) >= rank
                    pc = plsc.all_reduce_population_count(cond)
                    cand = jnp.where(pc > 0, pc + jnp.int32(j * L - 1),
                                     jnp.int32(-1))
                    bsel = jnp.maximum(bsel, cand)
                    carry = carry + csum
                acc = zeros_i
                for j in range(16):
                    t = tot_v[pl.ds(j * L, L)]
                    acc = acc + jnp.where(iota + j * L > bsel, t, 0)
                above = _splat(jnp.cumsum(acc), ilast)
                return bsel, above

            b1, above1 = scan(jnp.full((L,), jnp.int32(KTOP)))
            r1 = jnp.int32(KTOP) - above1

            def cp1(j, off):
                for u in range(8):
                    i = j * 8 + u
                    key = key_v[pl.ds(i * L, L)]
                    bkt = lax.shift_right_logical(
                        key ^ jnp.int32(INT_MIN32), 24)
                    m = bkt == b1
                    cs = jnp.cumsum(m.astype(jnp.int32))
                    plsc.store_scatter(comp_v, [off + cs - 1], key, mask=m)
                    off = off + plsc.all_reduce_population_count(m)
                return off

            c1 = lax.fori_loop(0, NV // 8, cp1, jnp.zeros((L,), jnp.int32))
            c1s = jnp.max(c1)
            nv2 = (c1s + L - 1) // L

            def h2(j, c):
                kk = comp_v[pl.ds(j * L, L)]
                valid = (iota + j * L) < c1
                bkt = lax.shift_right_logical(kk, 16) & jnp.int32(0xFF)
                plsc.addupdate_scatter(hist_v, [bkt + iota * NBKT], ones_i,
                                       mask=valid)
                return c

            lax.fori_loop(0, nv2, h2, 0)
            b2, above2 = scan(r1)
            r2 = r1 - above2

            def cp2(j, off):
                kk = comp_v[pl.ds(j * L, L)]
                valid = (iota + j * L) < c1
                bkt = lax.shift_right_logical(kk, 16) & jnp.int32(0xFF)
                m = (bkt == b2) & valid
                cs = jnp.cumsum(m.astype(jnp.int32))
                plsc.store_scatter(comp2_v, [off + cs - 1], kk, mask=m)
                return off + plsc.all_reduce_population_count(m)

            c2 = lax.fori_loop(0, nv2, cp2, jnp.zeros((L,), jnp.int32))
            nv3 = (jnp.max(c2) + L - 1) // L

            lo = jnp.zeros((L,), jnp.int32)
            for b in range(15, -1, -1):
                cand = lo + jnp.int32(1 << b)

                def cnt_body(j, a, cand=cand):
                    kk = comp2_v[pl.ds(j * L, L)]
                    lo16 = kk & jnp.int32(0xFFFF)
                    valid = (iota + j * L) < c2
                    hit = (lo16 >= cand) & valid
                    return a + plsc.all_reduce_population_count(hit)

                acc = lax.fori_loop(0, nv3, cnt_body, zeros_i)
                lo = jnp.where(acc >= r2, cand, lo)

            tk = ((b1 ^ jnp.int32(0x80)) << 24) | (b2 << 16) | lo
            vb = tk ^ (lax.shift_right_arithmetic(tk, 31)
                       & jnp.int32(0x7FFFFFFF))
            thrv = lax.bitcast_convert_type(vb, jnp.float32)

            def mk(j, c):
                for u in range(8):
                    i = j * 8 + u
                    v = row_v[pl.ds(rowoff + i * L, L)]
                    out_v[pl.ds(rowoff + i * L, L)] = jnp.where(
                        v >= thrv, jnp.maximum(v, 0.0), 0.0)
                return c

            lax.fori_loop(0, NV // 8, mk, 0)

        lax.fori_loop(0, rows // RB, do_batch, 0)

    mesh = plsc.VectorSubcoreMesh(core_axis_name="c", subcore_axis_name="s")
    return pl.kernel(
        body,
        mesh=mesh,
        compiler_params=pltpu.CompilerParams(needs_layout_passes=False),
        out_type=jax.ShapeDtypeStruct((RCHUNK * F,), jnp.float32),
        scratch_types=[
            pltpu.VMEM((4 * F,), jnp.float32),
            pltpu.VMEM((4 * F,), jnp.float32),
            pltpu.VMEM((F,), jnp.int32),
            pltpu.VMEM((NBKT * L,), jnp.int32),
            pltpu.VMEM((NBKT,), jnp.int32),
            pltpu.VMEM((F,), jnp.int32),
            pltpu.VMEM((F,), jnp.int32),
        ],
    )


def kernel(x, gamma, beta, W_enc, b_enc, W_dec, b_dec):
    B, T, _ = x.shape
    N = B * T
    DN = NT * H
    x2 = x.reshape(N, H)
    wh = W_enc.astype(jnp.bfloat16)
    g2 = gamma.reshape(1, H)
    bt2 = beta.reshape(1, H)
    be2 = b_enc.reshape(1, F)
    bd2 = b_dec.reshape(1, DN)
    wd16 = W_dec.astype(jnp.bfloat16)

    zp = pl.pallas_call(
        _enc_body,
        grid=(N // BM_ENC,),
        in_specs=[
            pl.BlockSpec((BM_ENC, H), lambda m: (m, 0)),
            pl.BlockSpec((1, H), lambda m: (0, 0)),
            pl.BlockSpec((1, H), lambda m: (0, 0)),
            pl.BlockSpec((H, F), lambda m: (0, 0)),
            pl.BlockSpec((1, F), lambda m: (0, 0)),
        ],
        out_specs=pl.BlockSpec((BM_ENC, F), lambda m: (m, 0)),
        out_shape=jax.ShapeDtypeStruct((N, F), jnp.float32),
    )(x2, g2, bt2, wh, be2)

    zp_flat = zp.reshape(N * F)
    ys = []
    zs = []
    for c in range(NCHUNK):
        z_c = _make_sc_select(c * RCHUNK)(zp_flat).reshape(RCHUNK, F)
        y_c = pl.pallas_call(
            _dec_body,
            grid=(DN // BN_DEC, RCHUNK // BM_DEC),
            in_specs=[
                pl.BlockSpec((BM_DEC, F), lambda n, m: (m, 0)),
                pl.BlockSpec((F, BN_DEC), lambda n, m: (0, n)),
                pl.BlockSpec((1, BN_DEC), lambda n, m: (0, n)),
            ],
            out_specs=pl.BlockSpec((BM_DEC, BN_DEC), lambda n, m: (m, n)),
            out_shape=jax.ShapeDtypeStruct((RCHUNK, DN), jnp.float32),
        )(z_c, wd16, bd2)
        zs.append(z_c)
        ys.append(y_c)

    y = jnp.concatenate(ys, axis=0)
    z = jnp.concatenate(zs, axis=0)
    return (y.reshape(B, T, NT, H), z.reshape(B, T, F))


# speedup vs baseline: 3.1720x; 3.1720x over previous
import jax
import jax.numpy as jnp
from jax.experimental import pallas as pl

H = 1024
F = 8192
NT = 2
KTOP = 983
BM_ENC = 256
BM_DEC = 256
BN_DEC = 1024
INT_MIN32 = -2147483648


def _enc_body(x_ref, g_ref, bt_ref, w_ref, be_ref, z_ref):
    x = x_ref[...]
    mean = jnp.mean(x, axis=1, keepdims=True)
    xc = x - mean
    var = jnp.mean(xc * xc, axis=1, keepdims=True)
    xn = xc * jax.lax.rsqrt(var + 1e-5)
    xn = xn * g_ref[...] + bt_ref[...]
    xh = xn.astype(jnp.bfloat16)
    zp = jnp.dot(xh, w_ref[...], preferred_element_type=jnp.float32)
    zp = zp + be_ref[...]
    zb = jax.lax.bitcast_convert_type(zp, jnp.int32)
    keys = jnp.where(zb < 0, zb ^ jnp.int32(0x7FFFFFFF), zb)
    cnt0 = jnp.sum((keys >= 0).astype(jnp.int32), axis=1, keepdims=True)
    lo0 = jnp.where(cnt0 >= KTOP, jnp.zeros_like(cnt0),
                    jnp.full_like(cnt0, jnp.int32(INT_MIN32)))

    def body(i, lo):
        cand = lo + jnp.left_shift(jnp.int32(1), 30 - i)
        cnt = jnp.sum((keys >= cand).astype(jnp.int32), axis=1, keepdims=True)
        return jnp.where(cnt >= KTOP, cand, lo)

    thr = jax.lax.fori_loop(0, 31, body, lo0)
    z_ref[...] = jnp.where(keys >= thr, jnp.maximum(zp, 0.0), 0.0)


def _dec_body(z_ref, w_ref, bd_ref, y_ref):
    zb16 = z_ref[...].astype(jnp.bfloat16)
    y = jnp.dot(zb16, w_ref[...], preferred_element_type=jnp.float32)
    y_ref[...] = y + bd_ref[...]


def kernel(x, gamma, beta, W_enc, b_enc, W_dec, b_dec):
    B, T, _ = x.shape
    N = B * T
    x2 = x.reshape(N, H)
    wh = W_enc.astype(jnp.bfloat16)
    g2 = gamma.reshape(1, H)
    bt2 = beta.reshape(1, H)
    be2 = b_enc.reshape(1, F)
    bd2 = b_dec.reshape(1, NT * H)
    wd16 = W_dec.astype(jnp.bfloat16)

    z = pl.pallas_call(
        _enc_body,
        grid=(N // BM_ENC,),
        in_specs=[
            pl.BlockSpec((BM_ENC, H), lambda m: (m, 0)),
            pl.BlockSpec((1, H), lambda m: (0, 0)),
            pl.BlockSpec((1, H), lambda m: (0, 0)),
            pl.BlockSpec((H, F), lambda m: (0, 0)),
            pl.BlockSpec((1, F), lambda m: (0, 0)),
        ],
        out_specs=pl.BlockSpec((BM_ENC, F), lambda m: (m, 0)),
        out_shape=jax.ShapeDtypeStruct((N, F), jnp.float32),
    )(x2, g2, bt2, wh, be2)

    DN = NT * H
    y = pl.pallas_call(
        _dec_body,
        grid=(DN // BN_DEC, N // BM_DEC),
        in_specs=[
            pl.BlockSpec((BM_DEC, F), lambda n, m: (m, 0)),
            pl.BlockSpec((F, BN_DEC), lambda n, m: (0, n)),
            pl.BlockSpec((1, BN_DEC), lambda n, m: (0, n)),
        ],
        out_specs=pl.BlockSpec((BM_DEC, BN_DEC), lambda n, m: (m, n)),
        out_shape=jax.ShapeDtypeStruct((N, DN), jnp.float32),
    )(z, wd16, bd2)

    return (y.reshape(B, T, NT, H), z.reshape(B, T, F))
